# Initial kernel scaffold; baseline (speedup 1.0000x reference)
import jax
import jax.numpy as jnp
from jax.experimental import pallas as pl


def kernel(verts, linked_ids):
    raise NotImplementedError("write your pallas kernel here")


# speedup vs baseline: 1.1416x; 1.1416x over previous
import functools

import jax
import jax.numpy as jnp
from jax import lax
from jax.experimental import pallas as pl
from jax.experimental.pallas import tpu as pltpu
from jax.experimental.pallas import tpu_sc as plsc

L = 16


def _build_sc_kernel(N, V, C):
    W = V * C
    n_workers = 32
    n_per = N // n_workers
    n_groups = -(-W // L)
    Wp = n_groups * L
    K = -(-V // L) * L
    KB = K + L
    map_sz = -(-(C * (V + 1)) // L) * L
    n_kgroups = K // L

    mesh = plsc.VectorSubcoreMesh(core_axis_name="c", subcore_axis_name="s")

    @functools.partial(
        pl.kernel,
        mesh=mesh,
        compiler_params=pltpu.CompilerParams(
            needs_layout_passes=False, use_tc_tiling_on_sc=False),
        out_type=jax.ShapeDtypeStruct((N, W), jnp.float32),
        scratch_types=[
            pltpu.VMEM((map_sz,), jnp.int32),
            pltpu.VMEM((KB,), jnp.int32),
            pltpu.VMEM((KB,), jnp.int32),
            pltpu.VMEM((map_sz,), jnp.float32),
            pltpu.VMEM((Wp,), jnp.float32),
        ],
    )
    def sc_kernel(verts_hbm, from_hbm, to_hbm, out_hbm,
                  col_map, from_v, to_v, in_buf, out_buf):
        wid = lax.axis_index("s") * 2 + lax.axis_index("c")

        pltpu.sync_copy(from_hbm, from_v)
        pltpu.sync_copy(to_hbm, to_v)

        lanes = lax.broadcasted_iota(jnp.int32, (L,), 0)

        def init_body(g, _):
            col_map[pl.ds(g * L, L)] = g * L + lanes
            return _
        lax.fori_loop(0, map_sz // L, init_body, None)

        def link_body(g, _):
            k0 = g * L
            f = from_v[pl.ds(k0, L)]
            t = to_v[pl.ds(k0, L)]
            dup = f != f
            for s in range(1, L):
                dup = jnp.logical_or(dup, f == from_v[pl.ds(k0 + s, L)])
            f3 = jnp.where(dup, jnp.int32(C * V), f * 3)
            t3 = t * 3
            plsc.store_scatter(col_map, [f3], t3)
            plsc.store_scatter(col_map, [f3 + 1], t3 + 1)
            plsc.store_scatter(col_map, [f3 + 2], t3 + 2)
            return _
        lax.fori_loop(0, n_kgroups, link_body, None)

        def batch_body(i, _):
            n = wid * n_per + i
            pltpu.sync_copy(verts_hbm.at[n], in_buf.at[pl.ds(0, W)])

            def g_body(g, _):
                q0 = g * L
                idx = col_map[pl.ds(q0, L)]
                out_buf[pl.ds(q0, L)] = plsc.load_gather(in_buf, [idx])
                return _
            lax.fori_loop(0, n_groups, g_body, None)

            pltpu.sync_copy(out_buf.at[pl.ds(0, W)], out_hbm.at[n])
            return _
        lax.fori_loop(0, n_per, batch_body, None)

    return sc_kernel


def kernel(verts, linked_ids):
    N, V, C = verts.shape
    K = -(-V // L) * L
    KB = K + L
    f = linked_ids[:, 0].astype(jnp.int32)
    t = linked_ids[:, 1].astype(jnp.int32)
    from_p = jnp.concatenate([
        f, jnp.full((K - V,), V, jnp.int32), jnp.full((L,), -1, jnp.int32)])
    to_p = jnp.concatenate([t, jnp.full((KB - V,), V, jnp.int32)])
    sc = _build_sc_kernel(N, V, C)
    out = sc(verts.reshape(N, V * C), from_p, to_p)
    return out.reshape(N, V, C)


# speedup vs baseline: 1.3857x; 1.2139x over previous
import functools

import jax
import jax.numpy as jnp
from jax import lax
from jax.experimental import pallas as pl
from jax.experimental.pallas import tpu as pltpu
from jax.experimental.pallas import tpu_sc as plsc

L = 16


def _build_sc_kernel(N, V, C):
    W = V * C
    n_workers = 32
    n_per = N // n_workers
    n_groups = -(-W // L)
    Wp = n_groups * L
    K = -(-V // L) * L
    KB = K + L
    map_sz = -(-(C * (V + 1)) // L) * L
    n_kgroups = K // L

    mesh = plsc.VectorSubcoreMesh(core_axis_name="c", subcore_axis_name="s")

    @functools.partial(
        pl.kernel,
        mesh=mesh,
        compiler_params=pltpu.CompilerParams(
            needs_layout_passes=False, use_tc_tiling_on_sc=False),
        out_type=jax.ShapeDtypeStruct((N, W), jnp.float32),
        scratch_types=[
            pltpu.VMEM((map_sz,), jnp.int32),
            pltpu.VMEM((KB,), jnp.int32),
            pltpu.VMEM((KB,), jnp.int32),
            pltpu.VMEM((map_sz,), jnp.float32),
            pltpu.VMEM((map_sz,), jnp.float32),
            pltpu.VMEM((Wp,), jnp.float32),
            pltpu.VMEM((Wp,), jnp.float32),
            pltpu.SemaphoreType.DMA,
            pltpu.SemaphoreType.DMA,
            pltpu.SemaphoreType.DMA,
            pltpu.SemaphoreType.DMA,
        ],
    )
    def sc_kernel(verts_hbm, from_hbm, to_hbm, out_hbm,
                  col_map, from_v, to_v, in0, in1, out0, out1,
                  sin0, sin1, sout0, sout1):
        wid = lax.axis_index("s") * 2 + lax.axis_index("c")

        pltpu.sync_copy(from_hbm, from_v)
        pltpu.sync_copy(to_hbm, to_v)

        lanes = lax.broadcasted_iota(jnp.int32, (L,), 0)

        def init_body(g, _):
            col_map[pl.ds(g * L, L)] = g * L + lanes
            return _
        lax.fori_loop(0, map_sz // L, init_body, None)

        def link_body(g, _):
            k0 = g * L
            f = from_v[pl.ds(k0, L)]
            t = to_v[pl.ds(k0, L)]
            dup = f != f
            for s in range(1, L):
                dup = jnp.logical_or(dup, f == from_v[pl.ds(k0 + s, L)])
            f3 = jnp.where(dup, jnp.int32(C * V), f * 3)
            t3 = t * 3
            plsc.store_scatter(col_map, [f3], t3)
            plsc.store_scatter(col_map, [f3 + 1], t3 + 1)
            plsc.store_scatter(col_map, [f3 + 2], t3 + 2)
            return _
        lax.fori_loop(0, n_kgroups, link_body, None)

        base = wid * n_per
        ibufs, obufs = (in0, in1), (out0, out1)
        isems, osems = (sin0, sin1), (sout0, sout1)

        def permute(ibuf, obuf):
            @plsc.parallel_loop(0, n_groups, unroll=8)
            def _(g):
                q0 = g * L
                idx = col_map[pl.ds(q0, L)]
                obuf[pl.ds(q0, L)] = plsc.load_gather(ibuf, [idx])

        pltpu.async_copy(verts_hbm.at[base], in0.at[pl.ds(0, W)], sin0)
        pltpu.async_copy(verts_hbm.at[base + 1], in1.at[pl.ds(0, W)], sin1)

        def ring_body(k, _):
            i = k * 2
            for b in range(2):
                n = base + i + b
                ibuf, obuf = ibufs[b], obufs[b]
                sin, sout = isems[b], osems[b]
                pltpu.make_async_copy(
                    verts_hbm.at[n], ibuf.at[pl.ds(0, W)], sin).wait()

                @pl.when(i + b >= 2)
                def _():
                    pltpu.make_async_copy(
                        obuf.at[pl.ds(0, W)], out_hbm.at[n - 2], sout).wait()

                permute(ibuf, obuf)
                pltpu.async_copy(obuf.at[pl.ds(0, W)], out_hbm.at[n], sout)

                @pl.when(i + b + 2 < n_per)
                def _():
                    pltpu.async_copy(
                        verts_hbm.at[n + 2], ibuf.at[pl.ds(0, W)], sin)
            return _
        lax.fori_loop(0, n_per // 2, ring_body, None)

        pltpu.make_async_copy(
            out0.at[pl.ds(0, W)], out_hbm.at[base + n_per - 2], sout0).wait()
        pltpu.make_async_copy(
            out1.at[pl.ds(0, W)], out_hbm.at[base + n_per - 1], sout1).wait()

    return sc_kernel


def kernel(verts, linked_ids):
    N, V, C = verts.shape
    K = -(-V // L) * L
    KB = K + L
    f = linked_ids[:, 0].astype(jnp.int32)
    t = linked_ids[:, 1].astype(jnp.int32)
    from_p = jnp.concatenate([
        f, jnp.full((K - V,), V, jnp.int32), jnp.full((L,), -1, jnp.int32)])
    to_p = jnp.concatenate([t, jnp.full((KB - V,), V, jnp.int32)])
    sc = _build_sc_kernel(N, V, C)
    out = sc(verts.reshape(N, V * C), from_p, to_p)
    return out.reshape(N, V, C)


# speedup vs baseline: 17.6248x; 12.7189x over previous
import functools

import jax
import jax.numpy as jnp
from jax import lax
from jax.experimental import pallas as pl
from jax.experimental.pallas import tpu as pltpu
from jax.experimental.pallas import tpu_sc as plsc

L = 16


def _build_sc_kernel(N, V, C):
    n_groups = -(-V // L)
    K = n_groups * L
    KB = K + 2 * L
    LT = N // 128
    n_cols = C * LT
    cols_per_sc = n_cols // 2
    CGRP = 108
    CH = CGRP * L
    CHB = CH + L
    chunk_groups = [CGRP, CGRP, CGRP, n_groups - 3 * CGRP]
    JS = -(-(-(-V // 16)) // 8) * 8
    J_LAST = ((V - JS) // 8) * 8
    TAIL = V - (J_LAST + JS)
    MS = JS + L
    MSB = MS + L
    GC = 48
    NQ = JS // GC

    mesh = plsc.VectorSubcoreMesh(core_axis_name="c", subcore_axis_name="s")

    @functools.partial(
        pl.kernel,
        mesh=mesh,
        compiler_params=pltpu.CompilerParams(
            needs_layout_passes=False, use_tc_tiling_on_sc=True),
        out_type=jax.ShapeDtypeStruct((C, V, N), jnp.float32),
        scratch_types=[
            pltpu.VMEM((MSB,), jnp.int32),
            pltpu.VMEM((CHB,), jnp.int32),
            pltpu.VMEM((CHB,), jnp.int32),
            pltpu.VMEM((GC, 128), jnp.float32),
            pltpu.VMEM((GC, 128), jnp.float32),
            pltpu.VMEM((8, 128), jnp.float32),
            pltpu.VMEM_SHARED((K, 128), jnp.float32),
            pltpu.VMEM_SHARED((K, 128), jnp.float32),
            pltpu.SemaphoreType.DMA,
            pltpu.SemaphoreType.DMA,
            pltpu.SemaphoreType.DMA,
            pltpu.SemaphoreType.DMA,
            pltpu.SemaphoreType.DMA,
            pltpu.SemaphoreType.DMA,
            pltpu.SemaphoreType.DMA,
            pltpu.SemaphoreType.DMA,
        ],
    )
    def sc_kernel(verts_hbm, from_hbm, to_hbm, out_hbm,
                  mslice, from_c, to_c, oh0, oh1, oht, colb0, colb1,
                  ssem0, ssem1, gsem0, gsem1, gsemt, osem0, osem1, osemt):
        sc = lax.axis_index("c")
        t = lax.axis_index("s")

        lanes = lax.broadcasted_iota(jnp.int32, (L,), 0)

        j0 = jnp.minimum(t * JS, jnp.int32(J_LAST))
        bufs = (colb0, colb1)
        ssems = (ssem0, ssem1)
        ohs = (oh0, oh1)
        gsems = (gsem0, gsem1)
        osems = (osem0, osem1)

        def col_nt(m):
            g = sc * cols_per_sc + m
            return g // LT, g % LT

        def stage(m, buf, ssem):
            c, nl = col_nt(m)
            pltpu.async_copy(
                verts_hbm.at[c, pl.ds(j0, JS), pl.ds(nl * 128, 128)],
                buf.at[pl.ds(j0, JS)], ssem)

            @pl.when(t == 15)
            def _():
                pltpu.async_copy(
                    verts_hbm.at[c, pl.ds(J_LAST + JS, TAIL),
                                 pl.ds(nl * 128, 128)],
                    buf.at[pl.ds(J_LAST + JS, TAIL)], ssem)

        def stage_wait(buf, ssem):
            pltpu.make_async_copy(
                verts_hbm.at[0, pl.ds(0, JS), pl.ds(0, 128)],
                buf.at[pl.ds(0, JS)], ssem).wait()

            @pl.when(t == 15)
            def _():
                pltpu.make_async_copy(
                    verts_hbm.at[0, pl.ds(0, TAIL), pl.ds(0, 128)],
                    buf.at[pl.ds(0, TAIL)], ssem).wait()

        stage(0, colb0, ssem0)
        stage(1, colb1, ssem1)

        def init_body(g, _):
            mslice[pl.ds(g * L, L)] = j0 + g * L + lanes
            return _
        lax.fori_loop(0, MSB // L, init_body, None)

        for ci, cgrp in enumerate(chunk_groups):
            pltpu.sync_copy(from_hbm.at[pl.ds(ci * CH, CHB)], from_c)
            pltpu.sync_copy(to_hbm.at[pl.ds(ci * CH, CHB)], to_c)

            def link_body(g, _):
                k0 = g * L
                f = from_c[pl.ds(k0, L)]
                tt = to_c[pl.ds(k0, L)]
                dup = f != f
                for s in range(1, L):
                    dup = jnp.logical_or(dup, f == from_c[pl.ds(k0 + s, L)])
                fl = f - j0
                bad = dup | (fl < 0) | (fl >= MS)
                fr = jnp.where(bad, jnp.int32(MS), fl)
                plsc.store_scatter(mslice, [fr], tt)
                return _
            lax.fori_loop(0, cgrp, link_body, None)

        idxs = [mslice.at[pl.ds(GC * q, GC)] for q in range(NQ)]
        idxt = mslice.at[pl.ds(JS, 8)]

        for m in range(cols_per_sc):
            c, nl = col_nt(m)
            buf, ssem = bufs[m % 2], ssems[m % 2]
            stage_wait(buf, ssem)
            plsc.subcore_barrier()

            for q in range(NQ):
                p = q % 2
                if m >= 1 or q >= 2:
                    pltpu.make_async_copy(
                        ohs[p], out_hbm.at[0, pl.ds(j0, GC), pl.ds(0, 128)],
                        osems[p]).wait()
                pltpu.async_copy(buf.at[idxs[q]], ohs[p], gsems[p])
                if q >= 1:
                    pltpu.make_async_copy(
                        buf.at[idxs[q - 1]], ohs[1 - p], gsems[1 - p]).wait()
                    pltpu.async_copy(
                        ohs[1 - p],
                        out_hbm.at[c, pl.ds(j0 + GC * (q - 1), GC),
                                   pl.ds(nl * 128, 128)], osems[1 - p])
            pl_last = (NQ - 1) % 2
            pltpu.make_async_copy(
                buf.at[idxs[NQ - 1]], ohs[pl_last], gsems[pl_last]).wait()
            pltpu.async_copy(
                ohs[pl_last],
                out_hbm.at[c, pl.ds(j0 + GC * (NQ - 1), GC),
                           pl.ds(nl * 128, 128)], osems[pl_last])

            if TAIL:
                @pl.when(t == 15)
                def _(m=m, c=c, nl=nl, buf=buf):
                    if m >= 1:
                        pltpu.make_async_copy(
                            oht.at[pl.ds(0, TAIL)],
                            out_hbm.at[0, pl.ds(0, TAIL), pl.ds(0, 128)],
                            osemt).wait()
                    pltpu.async_copy(buf.at[idxt], oht, gsemt).wait()
                    pltpu.async_copy(
                        oht.at[pl.ds(0, TAIL)],
                        out_hbm.at[c, pl.ds(J_LAST + JS, TAIL),
                                   pl.ds(nl * 128, 128)], osemt)

            plsc.subcore_barrier()
            if m + 2 < cols_per_sc:
                stage(m + 2, buf, ssem)

        pltpu.make_async_copy(
            oh0, out_hbm.at[0, pl.ds(j0, GC), pl.ds(0, 128)], osem0).wait()
        pltpu.make_async_copy(
            oh1, out_hbm.at[0, pl.ds(j0, GC), pl.ds(0, 128)], osem1).wait()
        if TAIL:
            @pl.when(t == 15)
            def _():
                pltpu.make_async_copy(
                    oht.at[pl.ds(0, TAIL)],
                    out_hbm.at[0, pl.ds(0, TAIL), pl.ds(0, 128)],
                    osemt).wait()

    return sc_kernel


def kernel(verts, linked_ids):
    N, V, C = verts.shape
    n_groups = -(-V // L)
    K = n_groups * L
    KB = K + 2 * L
    f = linked_ids[:, 0].astype(jnp.int32)
    t = linked_ids[:, 1].astype(jnp.int32)
    from_p = jnp.concatenate([f, jnp.full((KB - V,), V, jnp.int32)])
    to_p = jnp.concatenate([t, jnp.full((KB - V,), V, jnp.int32)])
    sc = _build_sc_kernel(N, V, C)
    out_t = sc(verts.transpose(2, 1, 0), from_p, to_p)
    return out_t.transpose(2, 1, 0)
